# baseline (speedup 1.0000x reference)
import jax
import jax.numpy as jnp
from jax.experimental import pallas as pl
from jax.experimental.pallas import tpu as pltpu

_N = 10000
_D_IN = 128
_D_OUT = 32
_BLOCK = 5000


def _gclstm_block(x_ref, h_ref, c_ref, w_ref, h_out_ref, c_out_ref):
    x = x_ref[:]
    hh = h_ref[:]
    c = c_ref[:]

    def gate(g):
        lo = g * _D_OUT
        hi = lo + _D_OUT
        return (jnp.dot(x, w_ref[0:_D_IN, lo:hi],
                        preferred_element_type=jnp.float32)
                + jnp.dot(hh, w_ref[_D_IN:_D_IN + _D_OUT, lo:hi],
                          preferred_element_type=jnp.float32)
                + w_ref[160:161, lo:hi] + w_ref[161:162, lo:hi])

    i_gate = jax.nn.sigmoid(gate(0) + w_ref[162:163, 0:_D_OUT] * c)
    f_gate = jax.nn.sigmoid(gate(1) + w_ref[162:163, _D_OUT:2 * _D_OUT] * c)
    t_cand = jnp.tanh(gate(2))
    c_new = f_gate * c + i_gate * t_cand
    o_gate = jax.nn.sigmoid(gate(3) + w_ref[162:163, 2 * _D_OUT:3 * _D_OUT] * c_new)
    h_out_ref[:] = o_gate * jnp.tanh(c_new)
    c_out_ref[:] = c_new


def kernel(x, edge_index, edge_weight, h, c,
           W_i, W_f, W_c, W_o, Th_i, Th_f, Th_c, Th_o,
           bconv_i, bconv_f, bconv_c, bconv_o,
           w_ci, w_cf, w_co, b_i, b_f, b_c, b_o):
    del edge_index, edge_weight

    w_all = jnp.concatenate([
        jnp.concatenate([W_i, W_f, W_c, W_o], axis=1),
        jnp.concatenate([Th_i, Th_f, Th_c, Th_o], axis=1),
        jnp.concatenate([bconv_i, bconv_f, bconv_c, bconv_o])[None, :],
        jnp.concatenate([b_i, b_f, b_c, b_o], axis=1),
        jnp.concatenate([w_ci, w_cf, w_co, w_co], axis=1),
    ], axis=0)

    row_spec = lambda i: (i, 0)
    h_new, c_new = pl.pallas_call(
        _gclstm_block,
        grid=(_N // _BLOCK,),
        in_specs=[
            pl.BlockSpec((_BLOCK, _D_IN), row_spec),
            pl.BlockSpec((_BLOCK, _D_OUT), row_spec),
            pl.BlockSpec((_BLOCK, _D_OUT), row_spec),
            pl.BlockSpec((163, 128), lambda i: (0, 0)),
        ],
        out_specs=[
            pl.BlockSpec((_BLOCK, _D_OUT), row_spec),
            pl.BlockSpec((_BLOCK, _D_OUT), row_spec),
        ],
        out_shape=[
            jax.ShapeDtypeStruct((_N, _D_OUT), jnp.float32),
            jax.ShapeDtypeStruct((_N, _D_OUT), jnp.float32),
        ],
        compiler_params=pltpu.CompilerParams(
            dimension_semantics=("arbitrary",),
        ),
    )(x, h, c, w_all)
    return (h_new, c_new)


# speedup vs baseline: 1.0393x; 1.0393x over previous
import jax
import jax.numpy as jnp
from jax.experimental import pallas as pl
from jax.experimental.pallas import tpu as pltpu

_N = 10000
_D_IN = 128
_D_OUT = 32
_BLOCK = 2000


def _gclstm_block(x_ref, h_ref, c_ref, w_ref, h_out_ref, c_out_ref):
    x = x_ref[:].astype(jnp.bfloat16)
    hh = h_ref[:].astype(jnp.bfloat16)
    c = c_ref[:]

    def gate(g):
        lo = g * _D_OUT
        hi = lo + _D_OUT
        return (jnp.dot(x, w_ref[0:_D_IN, lo:hi].astype(jnp.bfloat16),
                        preferred_element_type=jnp.float32)
                + jnp.dot(hh, w_ref[_D_IN:_D_IN + _D_OUT, lo:hi].astype(jnp.bfloat16),
                          preferred_element_type=jnp.float32)
                + w_ref[160:161, lo:hi] + w_ref[161:162, lo:hi])

    i_gate = jax.nn.sigmoid(gate(0) + w_ref[162:163, 0:_D_OUT] * c)
    f_gate = jax.nn.sigmoid(gate(1) + w_ref[162:163, _D_OUT:2 * _D_OUT] * c)
    t_cand = jnp.tanh(gate(2))
    c_new = f_gate * c + i_gate * t_cand
    o_gate = jax.nn.sigmoid(gate(3) + w_ref[162:163, 2 * _D_OUT:3 * _D_OUT] * c_new)
    h_out_ref[:] = o_gate * jnp.tanh(c_new)
    c_out_ref[:] = c_new


def kernel(x, edge_index, edge_weight, h, c,
           W_i, W_f, W_c, W_o, Th_i, Th_f, Th_c, Th_o,
           bconv_i, bconv_f, bconv_c, bconv_o,
           w_ci, w_cf, w_co, b_i, b_f, b_c, b_o):
    del edge_index, edge_weight

    w_all = jnp.concatenate([
        jnp.concatenate([W_i, W_f, W_c, W_o], axis=1),
        jnp.concatenate([Th_i, Th_f, Th_c, Th_o], axis=1),
        jnp.concatenate([bconv_i, bconv_f, bconv_c, bconv_o])[None, :],
        jnp.concatenate([b_i, b_f, b_c, b_o], axis=1),
        jnp.concatenate([w_ci, w_cf, w_co, w_co], axis=1),
    ], axis=0)

    row_spec = lambda i: (i, 0)
    h_new, c_new = pl.pallas_call(
        _gclstm_block,
        grid=(_N // _BLOCK,),
        in_specs=[
            pl.BlockSpec((_BLOCK, _D_IN), row_spec),
            pl.BlockSpec((_BLOCK, _D_OUT), row_spec),
            pl.BlockSpec((_BLOCK, _D_OUT), row_spec),
            pl.BlockSpec((163, 128), lambda i: (0, 0)),
        ],
        out_specs=[
            pl.BlockSpec((_BLOCK, _D_OUT), row_spec),
            pl.BlockSpec((_BLOCK, _D_OUT), row_spec),
        ],
        out_shape=[
            jax.ShapeDtypeStruct((_N, _D_OUT), jnp.float32),
            jax.ShapeDtypeStruct((_N, _D_OUT), jnp.float32),
        ],
        compiler_params=pltpu.CompilerParams(
            dimension_semantics=("arbitrary",),
        ),
    )(x, h, c, w_all)
    return (h_new, c_new)


# speedup vs baseline: 1.0801x; 1.0392x over previous
import jax
import jax.numpy as jnp
from jax.experimental import pallas as pl
from jax.experimental.pallas import tpu as pltpu

_N = 10000
_D_IN = 128
_D_OUT = 32
_BLOCK = 2000


def _gclstm_block(x_ref, h_ref, c_ref, w_ref, h_out_ref, c_out_ref):
    x = x_ref[:]
    hh = h_ref[:]
    c = c_ref[:]

    def gate(g):
        lo = g * _D_OUT
        hi = lo + _D_OUT
        return (jnp.dot(x, w_ref[0:_D_IN, lo:hi],
                        preferred_element_type=jnp.float32)
                + jnp.dot(hh, w_ref[_D_IN:_D_IN + _D_OUT, lo:hi],
                          preferred_element_type=jnp.float32)
                + w_ref[160:161, lo:hi] + w_ref[161:162, lo:hi])

    i_gate = jax.nn.sigmoid(gate(0) + w_ref[162:163, 0:_D_OUT] * c)
    f_gate = jax.nn.sigmoid(gate(1) + w_ref[162:163, _D_OUT:2 * _D_OUT] * c)
    t_cand = jnp.tanh(gate(2))
    c_new = f_gate * c + i_gate * t_cand
    o_gate = jax.nn.sigmoid(gate(3) + w_ref[162:163, 2 * _D_OUT:3 * _D_OUT] * c_new)
    h_out_ref[:] = o_gate * jnp.tanh(c_new)
    c_out_ref[:] = c_new


def kernel(x, edge_index, edge_weight, h, c,
           W_i, W_f, W_c, W_o, Th_i, Th_f, Th_c, Th_o,
           bconv_i, bconv_f, bconv_c, bconv_o,
           w_ci, w_cf, w_co, b_i, b_f, b_c, b_o):
    del edge_index, edge_weight

    w_all = jnp.concatenate([
        jnp.concatenate([W_i, W_f, W_c, W_o], axis=1),
        jnp.concatenate([Th_i, Th_f, Th_c, Th_o], axis=1),
        jnp.concatenate([bconv_i, bconv_f, bconv_c, bconv_o])[None, :],
        jnp.concatenate([b_i, b_f, b_c, b_o], axis=1),
        jnp.concatenate([w_ci, w_cf, w_co, w_co], axis=1),
    ], axis=0)

    row_spec = lambda i: (i, 0)
    h_new, c_new = pl.pallas_call(
        _gclstm_block,
        grid=(_N // _BLOCK,),
        in_specs=[
            pl.BlockSpec((_BLOCK, _D_IN), row_spec),
            pl.BlockSpec((_BLOCK, _D_OUT), row_spec),
            pl.BlockSpec((_BLOCK, _D_OUT), row_spec),
            pl.BlockSpec((163, 128), lambda i: (0, 0)),
        ],
        out_specs=[
            pl.BlockSpec((_BLOCK, _D_OUT), row_spec),
            pl.BlockSpec((_BLOCK, _D_OUT), row_spec),
        ],
        out_shape=[
            jax.ShapeDtypeStruct((_N, _D_OUT), jnp.float32),
            jax.ShapeDtypeStruct((_N, _D_OUT), jnp.float32),
        ],
        compiler_params=pltpu.CompilerParams(
            dimension_semantics=("arbitrary",),
        ),
    )(x, h, c, w_all)
    return (h_new, c_new)
